# baseline (speedup 1.0000x reference)
import functools

import jax
import jax.numpy as jnp
from jax import lax
from jax.experimental import pallas as pl
from jax.experimental.pallas import tpu as pltpu
from jax.experimental.pallas import tpu_sc as plsc

EMB = 10
LANES = 16


@functools.lru_cache(maxsize=None)
def _build(B, C):
    info = plsc.get_sparse_core_info()
    nw = info.num_cores * info.num_subcores
    assert B % (LANES * nw) == 0 and C >= 8
    cpw = B // nw

    cpc = B // info.num_cores
    mesh = plsc.ScalarSubcoreMesh(axis_name="c")

    @functools.partial(
        pl.kernel,
        mesh=mesh,
        out_type=jax.ShapeDtypeStruct((EMB, B), jnp.int32),
        scratch_types=[
            pltpu.VMEM_SHARED((cpc,), jnp.int32),
            pltpu.SemaphoreType.DMA,
        ],
        compiler_params=pltpu.CompilerParams(
            needs_layout_passes=False,
            skip_device_barrier=True,
            disable_bounds_checks=True,
            disable_semaphore_checks=True,
        ),
    )
    def run(in_hbm, out_hbm, col_v, sem):
        cb = lax.axis_index("c") * cpc
        src = in_hbm.at[0, pl.ds(cb, cpc)]
        cps = [
            pltpu.async_copy(src, out_hbm.at[j, pl.ds(cb, cpc)], sem)
            for j in range(EMB)
        ]
        for cp in cps:
            cp.wait()

    return run


def kernel(inputs):
    B, C = inputs.shape
    return _build(B, C)(inputs.astype(jnp.int32).T).T


# speedup vs baseline: 1.9950x; 1.9950x over previous
import functools

import jax
import jax.numpy as jnp
from jax import lax
from jax.experimental import pallas as pl
from jax.experimental.pallas import tpu as pltpu
from jax.experimental.pallas import tpu_sc as plsc

EMB = 10
LANES = 16


@functools.lru_cache(maxsize=None)
def _build(B, C):
    info = plsc.get_sparse_core_info()
    nw = info.num_cores * info.num_subcores
    assert B % (LANES * nw) == 0 and C >= 8
    cpw = B // nw

    cpc = B // info.num_cores
    mesh = plsc.ScalarSubcoreMesh(axis_name="c")

    @functools.partial(
        pl.kernel,
        mesh=mesh,
        out_type=jax.ShapeDtypeStruct((EMB, B), jnp.int32),
        scratch_types=[
            pltpu.VMEM_SHARED((cpc,), jnp.int32),
            pltpu.SemaphoreType.DMA,
        ],
        compiler_params=pltpu.CompilerParams(
            needs_layout_passes=False,
            skip_device_barrier=True,
            disable_bounds_checks=True,
            disable_semaphore_checks=True,
        ),
    )
    def run(in_hbm, out_hbm, col_v, sem):
        cb = lax.axis_index("c") * cpc
        pltpu.sync_copy(in_hbm.at[0, pl.ds(cb, cpc)], col_v)
        cps = [
            pltpu.async_copy(col_v, out_hbm.at[j, pl.ds(cb, cpc)], sem)
            for j in range(EMB)
        ]
        for cp in cps:
            cp.wait()

    return run


def kernel(inputs):
    B, C = inputs.shape
    return _build(B, C)(inputs.astype(jnp.int32).T).T


# speedup vs baseline: 2.0035x; 1.0042x over previous
import functools

import jax
import jax.numpy as jnp
from jax import lax
from jax.experimental import pallas as pl
from jax.experimental.pallas import tpu as pltpu
from jax.experimental.pallas import tpu_sc as plsc

EMB = 10


@functools.lru_cache(maxsize=None)
def _build(B, C):
    info = plsc.get_sparse_core_info()
    assert B % (8 * info.num_cores) == 0
    cpc = B // info.num_cores

    @functools.partial(
        pl.kernel,
        mesh=plsc.ScalarSubcoreMesh(axis_name="c"),
        out_type=jax.ShapeDtypeStruct((EMB, B), jnp.int32),
        scratch_types=[
            pltpu.VMEM_SHARED((cpc,), jnp.int32),
            pltpu.SemaphoreType.DMA,
        ],
        compiler_params=pltpu.CompilerParams(
            needs_layout_passes=False,
            skip_device_barrier=True,
            disable_bounds_checks=True,
            disable_semaphore_checks=True,
        ),
    )
    def run(in_hbm, out_hbm, col_v, sem):
        cb = lax.axis_index("c") * cpc
        pltpu.sync_copy(in_hbm.at[0, pl.ds(cb, cpc)], col_v)
        cps = [
            pltpu.async_copy(col_v, out_hbm.at[j, pl.ds(cb, cpc)], sem)
            for j in range(EMB)
        ]
        for cp in cps:
            cp.wait()

    return run


def kernel(inputs):
    B, C = inputs.shape
    return _build(B, C)(inputs.astype(jnp.int32).T).T
